# initial kernel scaffold (unmeasured)
import jax
import jax.numpy as jnp
from jax import lax
from jax.experimental import pallas as pl
from jax.experimental.pallas import tpu as pltpu

N_DEV = 32


def _gelu(y):
    c = 0.7978845608028654
    return 0.5 * y * (1.0 + jnp.tanh(c * (y + 0.044715 * y * y * y)))


def kernel(x, w_mat):
    m, k_per = x.shape
    _, n = w_mat.shape
    mc = m // N_DEV

    def body(x_ref, w_ref, out_ref, send_ref, recv_ref,
             send_sem, recv_sem, copy_sem, credit_sem):
        d = lax.axis_index("i")
        left = (d - 1) % N_DEV
        right = (d + 1) % N_DEV

        barrier = pltpu.get_barrier_semaphore()
        for nbr in (left, right):
            pl.semaphore_signal(barrier, inc=1, device_id=(nbr,),
                                device_id_type=pl.DeviceIdType.MESH)
        pl.semaphore_wait(barrier, 2)

        def mm(c):
            return jnp.dot(x_ref[pl.ds(c * mc, mc), :], w_ref[...],
                           preferred_element_type=jnp.float32)

        def hop():
            rdma = pltpu.make_async_remote_copy(
                src_ref=send_ref, dst_ref=recv_ref,
                send_sem=send_sem, recv_sem=recv_sem,
                device_id=(right,), device_id_type=pl.DeviceIdType.MESH)
            rdma.start()
            rdma.wait_recv()
            rdma.wait_send()

        def give_credit():
            pl.semaphore_signal(credit_sem, inc=1, device_id=(left,),
                                device_id_type=pl.DeviceIdType.MESH)

        send_ref[...] = mm(d % N_DEV)

        def rs_body(s, carry):
            @pl.when(s > 0)
            def _():
                pl.semaphore_wait(credit_sem, 1)
            hop()
            send_ref[...] = recv_ref[...] + mm((d - s - 1) % N_DEV)
            give_credit()
            return carry

        lax.fori_loop(0, N_DEV - 1, rs_body, 0)

        send_ref[...] = _gelu(send_ref[...])
        own = (d + 1) % N_DEV
        cp = pltpu.make_async_copy(
            send_ref, out_ref.at[pl.ds(own * mc, mc), :], copy_sem)
        cp.start()
        cp.wait()

        def ag_body(t, carry):
            pl.semaphore_wait(credit_sem, 1)
            hop()
            origin = (d - t) % N_DEV
            cp = pltpu.make_async_copy(
                recv_ref, out_ref.at[pl.ds(origin * mc, mc), :], copy_sem)
            cp.start()
            cp.wait()

            @pl.when(t < N_DEV - 2)
            def _():
                send_ref[...] = recv_ref[...]
                give_credit()
            return carry

        lax.fori_loop(0, N_DEV - 1, ag_body, 0)

    return pl.pallas_call(
        body,
        out_shape=jax.ShapeDtypeStruct((m, n), jnp.float32),
        in_specs=[
            pl.BlockSpec(memory_space=pltpu.VMEM),
            pl.BlockSpec(memory_space=pltpu.VMEM),
        ],
        out_specs=pl.BlockSpec(memory_space=pltpu.ANY),
        scratch_shapes=[
            pltpu.VMEM((mc, n), jnp.float32),
            pltpu.VMEM((mc, n), jnp.float32),
            pltpu.SemaphoreType.DMA,
            pltpu.SemaphoreType.DMA,
            pltpu.SemaphoreType.DMA,
            pltpu.SemaphoreType.REGULAR,
        ],
        compiler_params=pltpu.CompilerParams(collective_id=0),
    )(x, w_mat)


# baseline (device time: 3411283 ns/iter reference)
import jax
import jax.numpy as jnp
from jax import lax
from jax.experimental import pallas as pl
from jax.experimental.pallas import tpu as pltpu

N_DEV = 32


def _gelu(y):
    c = 0.7978845608028654
    return 0.5 * y * (1.0 + jnp.tanh(c * (y + 0.044715 * y * y * y)))


def kernel(x, w_mat):
    m, k_per = x.shape
    _, n = w_mat.shape
    mc = m // N_DEV

    def body(x_ref, w_ref, out_ref, send_ref, recv_ref,
             send_sem, recv_sem, copy_sem, credit_sem):
        d = lax.axis_index("i")
        left = (d - 1) % N_DEV
        right = (d + 1) % N_DEV

        barrier = pltpu.get_barrier_semaphore()
        for nbr in (left, right):
            pl.semaphore_signal(barrier, inc=1, device_id=(nbr,),
                                device_id_type=pl.DeviceIdType.MESH)
        pl.semaphore_wait(barrier, 2)

        def mm(c):
            return jnp.dot(x_ref[pl.ds(c * mc, mc), :], w_ref[...],
                           preferred_element_type=jnp.float32)

        def hop():
            rdma = pltpu.make_async_remote_copy(
                src_ref=send_ref, dst_ref=recv_ref,
                send_sem=send_sem, recv_sem=recv_sem,
                device_id=(right,), device_id_type=pl.DeviceIdType.MESH)
            rdma.start()
            rdma.wait_recv()
            rdma.wait_send()

        def give_credit():
            pl.semaphore_signal(credit_sem, inc=1, device_id=(left,),
                                device_id_type=pl.DeviceIdType.MESH)

        send_ref[...] = mm(d % N_DEV)

        def rs_body(s, carry):
            @pl.when(s > 0)
            def _():
                pl.semaphore_wait(credit_sem, 1)
            hop()
            send_ref[...] = recv_ref[...] + mm((d - s - 1) % N_DEV)
            give_credit()
            return carry

        lax.fori_loop(0, N_DEV - 1, rs_body, 0)

        send_ref[...] = _gelu(send_ref[...])
        own = (d + 1) % N_DEV
        cp = pltpu.make_async_copy(
            send_ref, out_ref.at[pl.ds(own * mc, mc), :], copy_sem)
        cp.start()
        cp.wait()

        def ag_body(t, carry):
            pl.semaphore_wait(credit_sem, 1)
            hop()
            origin = (d - t) % N_DEV
            cp = pltpu.make_async_copy(
                recv_ref, out_ref.at[pl.ds(origin * mc, mc), :], copy_sem)
            cp.start()
            cp.wait()

            @pl.when(t < N_DEV - 2)
            def _():
                send_ref[...] = recv_ref[...]
                give_credit()
            return carry

        lax.fori_loop(0, N_DEV - 1, ag_body, 0)

    return pl.pallas_call(
        body,
        out_shape=jax.ShapeDtypeStruct((m, n), jnp.float32),
        in_specs=[
            pl.BlockSpec(memory_space=pltpu.VMEM),
            pl.BlockSpec(memory_space=pltpu.VMEM),
        ],
        out_specs=pl.BlockSpec(memory_space=pl.ANY),
        scratch_shapes=[
            pltpu.VMEM((mc, n), jnp.float32),
            pltpu.VMEM((mc, n), jnp.float32),
            pltpu.SemaphoreType.DMA,
            pltpu.SemaphoreType.DMA,
            pltpu.SemaphoreType.DMA,
            pltpu.SemaphoreType.REGULAR,
        ],
        compiler_params=pltpu.CompilerParams(collective_id=0),
    )(x, w_mat)


# device time: 3102742 ns/iter; 1.0994x vs baseline; 1.0994x over previous
import jax
import jax.numpy as jnp
from jax import lax
from jax.experimental import pallas as pl
from jax.experimental.pallas import tpu as pltpu

N_DEV = 32


def _gelu(y):
    c = 0.7978845608028654
    return 0.5 * y * (1.0 + jnp.tanh(c * (y + 0.044715 * y * y * y)))


def kernel(x, w_mat):
    m, k_per = x.shape
    _, n = w_mat.shape
    mc = m // N_DEV
    mh = mc // 2

    def body(x_ref, w_ref, out_ref,
             send_a, recv_a, send_b, recv_b, local_a, local_b,
             ssem_a, rsem_a, ssem_b, rsem_b, csem_a, csem_b,
             credit_a, credit_b):
        d = lax.axis_index("i")
        left = (d - 1) % N_DEV
        right = (d + 1) % N_DEV

        barrier = pltpu.get_barrier_semaphore()
        for nbr in (left, right):
            pl.semaphore_signal(barrier, inc=1, device_id=(nbr,),
                                device_id_type=pl.DeviceIdType.MESH)
        pl.semaphore_wait(barrier, 2)

        def mm_a(c):
            return jnp.dot(x_ref[pl.ds(c * mc, mh), :], w_ref[...],
                           preferred_element_type=jnp.float32)

        def mm_b(c):
            return jnp.dot(x_ref[pl.ds(c * mc + mh, mh), :], w_ref[...],
                           preferred_element_type=jnp.float32)

        def start_hops():
            ra = pltpu.make_async_remote_copy(
                src_ref=send_a, dst_ref=recv_a, send_sem=ssem_a,
                recv_sem=rsem_a, device_id=(right,),
                device_id_type=pl.DeviceIdType.MESH)
            rb = pltpu.make_async_remote_copy(
                src_ref=send_b, dst_ref=recv_b, send_sem=ssem_b,
                recv_sem=rsem_b, device_id=(left,),
                device_id_type=pl.DeviceIdType.MESH)
            ra.start()
            rb.start()
            return ra, rb

        def give_credits():
            pl.semaphore_signal(credit_a, inc=1, device_id=(left,),
                                device_id_type=pl.DeviceIdType.MESH)
            pl.semaphore_signal(credit_b, inc=1, device_id=(right,),
                                device_id_type=pl.DeviceIdType.MESH)

        def wait_credits():
            pl.semaphore_wait(credit_a, 1)
            pl.semaphore_wait(credit_b, 1)

        send_a[...] = mm_a(d % N_DEV)
        send_b[...] = mm_b(d % N_DEV)

        def rs_body(s, carry):
            @pl.when(s > 0)
            def _():
                wait_credits()
            ra, rb = start_hops()
            local_a[...] = mm_a((d - s - 1) % N_DEV)
            local_b[...] = mm_b((d + s + 1) % N_DEV)
            ra.wait_recv()
            rb.wait_recv()
            ra.wait_send()
            rb.wait_send()
            send_a[...] = recv_a[...] + local_a[...]
            send_b[...] = recv_b[...] + local_b[...]
            give_credits()
            return carry

        lax.fori_loop(0, N_DEV - 1, rs_body, 0)

        send_a[...] = _gelu(send_a[...])
        send_b[...] = _gelu(send_b[...])
        own_a = (d + 1) % N_DEV
        own_b = (d - 1) % N_DEV
        cp_a = pltpu.make_async_copy(
            send_a, out_ref.at[pl.ds(own_a * mc, mh), :], csem_a)
        cp_b = pltpu.make_async_copy(
            send_b, out_ref.at[pl.ds(own_b * mc + mh, mh), :], csem_b)
        cp_a.start()
        cp_b.start()
        cp_a.wait()
        cp_b.wait()

        def ag_body(t, carry):
            wait_credits()
            ra, rb = start_hops()
            ra.wait_recv()
            rb.wait_recv()
            origin_a = (d - t) % N_DEV
            origin_b = (d + t) % N_DEV
            cp_a = pltpu.make_async_copy(
                recv_a, out_ref.at[pl.ds(origin_a * mc, mh), :], csem_a)
            cp_b = pltpu.make_async_copy(
                recv_b, out_ref.at[pl.ds(origin_b * mc + mh, mh), :], csem_b)
            cp_a.start()
            cp_b.start()
            ra.wait_send()
            rb.wait_send()

            @pl.when(t < N_DEV - 2)
            def _():
                send_a[...] = recv_a[...]
                send_b[...] = recv_b[...]
            cp_a.wait()
            cp_b.wait()

            @pl.when(t < N_DEV - 2)
            def _():
                give_credits()
            return carry

        lax.fori_loop(0, N_DEV - 1, ag_body, 0)

    return pl.pallas_call(
        body,
        out_shape=jax.ShapeDtypeStruct((m, n), jnp.float32),
        in_specs=[
            pl.BlockSpec(memory_space=pltpu.VMEM),
            pl.BlockSpec(memory_space=pltpu.VMEM),
        ],
        out_specs=pl.BlockSpec(memory_space=pl.ANY),
        scratch_shapes=[
            pltpu.VMEM((mh, n), jnp.float32),
            pltpu.VMEM((mh, n), jnp.float32),
            pltpu.VMEM((mh, n), jnp.float32),
            pltpu.VMEM((mh, n), jnp.float32),
            pltpu.VMEM((mh, n), jnp.float32),
            pltpu.VMEM((mh, n), jnp.float32),
            pltpu.SemaphoreType.DMA,
            pltpu.SemaphoreType.DMA,
            pltpu.SemaphoreType.DMA,
            pltpu.SemaphoreType.DMA,
            pltpu.SemaphoreType.DMA,
            pltpu.SemaphoreType.DMA,
            pltpu.SemaphoreType.REGULAR,
            pltpu.SemaphoreType.REGULAR,
        ],
        compiler_params=pltpu.CompilerParams(collective_id=0),
    )(x, w_mat)


# device time: 1721973 ns/iter; 1.9810x vs baseline; 1.8019x over previous
import jax
import jax.numpy as jnp
from jax import lax
from jax.experimental import pallas as pl
from jax.experimental.pallas import tpu as pltpu

N_DEV = 32

_SNAKE_XY = {(0, 0): 0, (1, 0): 1, (1, 1): 2, (0, 1): 3,
             (0, 2): 4, (1, 2): 5, (1, 3): 6, (0, 3): 7}
_COORD_TO_LOGICAL = {(x, y, z): 8 * z + f
                     for (x, y), f in _SNAKE_XY.items() for z in range(4)}

_C16 = [(0, 0), (0, 1), (0, 2), (0, 3), (1, 3), (1, 2), (1, 1), (2, 1),
        (2, 2), (2, 3), (3, 3), (3, 2), (3, 1), (3, 0), (2, 0), (1, 0)]
_CYCLE = [(0, y, z) for (y, z) in _C16] + [(1, y, z) for (y, z) in _C16[::-1]]

assert len(set(_CYCLE)) == N_DEV
for _a, _b in zip(_CYCLE, _CYCLE[1:] + _CYCLE[:1]):
    assert sum(abs(i - j) for i, j in zip(_a, _b)) == 1, (_a, _b)

RING = [_COORD_TO_LOGICAL[c] for c in _CYCLE]
POS = [0] * N_DEV
for _p, _l in enumerate(RING):
    POS[_l] = _p


def _gelu(y):
    c = 0.7978845608028654
    return 0.5 * y * (1.0 + jnp.tanh(c * (y + 0.044715 * y * y * y)))


def kernel(x, w_mat):
    m, k_per = x.shape
    _, n = w_mat.shape
    mc = m // N_DEV
    mh = mc // 2

    d = lax.axis_index("i")
    ring_t = jnp.asarray(RING, dtype=jnp.int32)
    pos_t = jnp.asarray(POS, dtype=jnp.int32)
    p = pos_t[d]
    right = ring_t[(p + 1) % N_DEV]
    left = ring_t[(p - 1) % N_DEV]
    meta = jnp.stack([p, left, right]).astype(jnp.int32)

    def body(x_ref, w_ref, meta_ref, out_ref,
             send_a, recv_a, send_b, recv_b, local_a, local_b,
             ssem_a, rsem_a, ssem_b, rsem_b, csem_a, csem_b,
             credit_a, credit_b):
        p = meta_ref[0]
        left = meta_ref[1]
        right = meta_ref[2]

        barrier = pltpu.get_barrier_semaphore()
        for nbr in (left, right):
            pl.semaphore_signal(barrier, inc=1, device_id=(nbr,),
                                device_id_type=pl.DeviceIdType.MESH)
        pl.semaphore_wait(barrier, 2)

        def mm_a(c):
            return jnp.dot(x_ref[pl.ds(c * mc, mh), :], w_ref[...],
                           preferred_element_type=jnp.float32)

        def mm_b(c):
            return jnp.dot(x_ref[pl.ds(c * mc + mh, mh), :], w_ref[...],
                           preferred_element_type=jnp.float32)

        def start_hops():
            ra = pltpu.make_async_remote_copy(
                src_ref=send_a, dst_ref=recv_a, send_sem=ssem_a,
                recv_sem=rsem_a, device_id=(right,),
                device_id_type=pl.DeviceIdType.MESH)
            rb = pltpu.make_async_remote_copy(
                src_ref=send_b, dst_ref=recv_b, send_sem=ssem_b,
                recv_sem=rsem_b, device_id=(left,),
                device_id_type=pl.DeviceIdType.MESH)
            ra.start()
            rb.start()
            return ra, rb

        def give_credits():
            pl.semaphore_signal(credit_a, inc=1, device_id=(left,),
                                device_id_type=pl.DeviceIdType.MESH)
            pl.semaphore_signal(credit_b, inc=1, device_id=(right,),
                                device_id_type=pl.DeviceIdType.MESH)

        def wait_credits():
            pl.semaphore_wait(credit_a, 1)
            pl.semaphore_wait(credit_b, 1)

        send_a[...] = mm_a(p % N_DEV)
        send_b[...] = mm_b(p % N_DEV)

        def rs_body(s, carry):
            @pl.when(s > 0)
            def _():
                wait_credits()
            ra, rb = start_hops()
            local_a[...] = mm_a((p - s - 1) % N_DEV)
            local_b[...] = mm_b((p + s + 1) % N_DEV)
            ra.wait_recv()
            rb.wait_recv()
            ra.wait_send()
            rb.wait_send()
            send_a[...] = recv_a[...] + local_a[...]
            send_b[...] = recv_b[...] + local_b[...]
            give_credits()
            return carry

        lax.fori_loop(0, N_DEV - 1, rs_body, 0)

        send_a[...] = _gelu(send_a[...])
        send_b[...] = _gelu(send_b[...])
        own_a = (p + 1) % N_DEV
        own_b = (p - 1) % N_DEV
        cp_a = pltpu.make_async_copy(
            send_a, out_ref.at[pl.ds(own_a * mc, mh), :], csem_a)
        cp_b = pltpu.make_async_copy(
            send_b, out_ref.at[pl.ds(own_b * mc + mh, mh), :], csem_b)
        cp_a.start()
        cp_b.start()
        cp_a.wait()
        cp_b.wait()

        def ag_body(t, carry):
            wait_credits()
            ra, rb = start_hops()
            ra.wait_recv()
            rb.wait_recv()
            origin_a = (p - t) % N_DEV
            origin_b = (p + t) % N_DEV
            cp_a = pltpu.make_async_copy(
                recv_a, out_ref.at[pl.ds(origin_a * mc, mh), :], csem_a)
            cp_b = pltpu.make_async_copy(
                recv_b, out_ref.at[pl.ds(origin_b * mc + mh, mh), :], csem_b)
            cp_a.start()
            cp_b.start()
            ra.wait_send()
            rb.wait_send()

            @pl.when(t < N_DEV - 2)
            def _():
                send_a[...] = recv_a[...]
                send_b[...] = recv_b[...]
            cp_a.wait()
            cp_b.wait()

            @pl.when(t < N_DEV - 2)
            def _():
                give_credits()
            return carry

        lax.fori_loop(0, N_DEV - 1, ag_body, 0)

    return pl.pallas_call(
        body,
        out_shape=jax.ShapeDtypeStruct((m, n), jnp.float32),
        in_specs=[
            pl.BlockSpec(memory_space=pltpu.VMEM),
            pl.BlockSpec(memory_space=pltpu.VMEM),
            pl.BlockSpec(memory_space=pltpu.SMEM),
        ],
        out_specs=pl.BlockSpec(memory_space=pl.ANY),
        scratch_shapes=[
            pltpu.VMEM((mh, n), jnp.float32),
            pltpu.VMEM((mh, n), jnp.float32),
            pltpu.VMEM((mh, n), jnp.float32),
            pltpu.VMEM((mh, n), jnp.float32),
            pltpu.VMEM((mh, n), jnp.float32),
            pltpu.VMEM((mh, n), jnp.float32),
            pltpu.SemaphoreType.DMA,
            pltpu.SemaphoreType.DMA,
            pltpu.SemaphoreType.DMA,
            pltpu.SemaphoreType.DMA,
            pltpu.SemaphoreType.DMA,
            pltpu.SemaphoreType.DMA,
            pltpu.SemaphoreType.REGULAR,
            pltpu.SemaphoreType.REGULAR,
        ],
        compiler_params=pltpu.CompilerParams(collective_id=0),
    )(x, w_mat, meta)


# device time: 1497267 ns/iter; 2.2783x vs baseline; 1.1501x over previous
import jax
import jax.numpy as jnp
from jax import lax
from jax.experimental import pallas as pl
from jax.experimental.pallas import tpu as pltpu

N_DEV = 32
N_STEP = 2 * N_DEV - 2

_SNAKE_XY = {(0, 0): 0, (1, 0): 1, (1, 1): 2, (0, 1): 3,
             (0, 2): 4, (1, 2): 5, (1, 3): 6, (0, 3): 7}
_COORD_TO_LOGICAL = {(x, y, z): 8 * z + f
                     for (x, y), f in _SNAKE_XY.items() for z in range(4)}

_C16 = [(0, 0), (0, 1), (0, 2), (0, 3), (1, 3), (1, 2), (1, 1), (2, 1),
        (2, 2), (2, 3), (3, 3), (3, 2), (3, 1), (3, 0), (2, 0), (1, 0)]
_CYCLE = [(0, y, z) for (y, z) in _C16] + [(1, y, z) for (y, z) in _C16[::-1]]

assert len(set(_CYCLE)) == N_DEV
for _a, _b in zip(_CYCLE, _CYCLE[1:] + _CYCLE[:1]):
    assert sum(abs(i - j) for i, j in zip(_a, _b)) == 1, (_a, _b)

RING = [_COORD_TO_LOGICAL[c] for c in _CYCLE]
POS = [0] * N_DEV
for _p, _l in enumerate(RING):
    POS[_l] = _p


def _gelu(y):
    c = 0.7978845608028654
    return 0.5 * y * (1.0 + jnp.tanh(c * (y + 0.044715 * y * y * y)))


def kernel(x, w_mat):
    m, k_per = x.shape
    _, n = w_mat.shape
    mc = m // N_DEV
    mh = mc // 2
    nh = n // 2

    d = lax.axis_index("i")
    ring_t = jnp.asarray(RING, dtype=jnp.int32)
    pos_t = jnp.asarray(POS, dtype=jnp.int32)
    p = pos_t[d]
    right = ring_t[(p + 1) % N_DEV]
    left = ring_t[(p - 1) % N_DEV]
    meta = jnp.stack([p, left, right]).astype(jnp.int32)

    def body(x_ref, w_ref, meta_ref, out_ref, send_ref, recv_ref,
             ssem, rsem, csem, credit):
        p = meta_ref[0]
        left = meta_ref[1]
        right = meta_ref[2]

        lane_dst = [right, right, left, left]
        lane_up = [left, left, right, right]

        def lane_row0(lane, g):
            if lane < 2:
                return ((p - g - 1) % N_DEV) * mc
            return ((p + g + 1) % N_DEV) * mc + mh

        def lane_col0(lane):
            return (lane % 2) * nh

        def mm(lane, g):
            c0 = lane_col0(lane)
            return jnp.dot(x_ref[pl.ds(lane_row0(lane, g), mh), :],
                           w_ref[:, c0:c0 + nh],
                           preferred_element_type=jnp.float32)

        def mk_rdma(lane, src, dst_slot):
            return pltpu.make_async_remote_copy(
                src_ref=src, dst_ref=recv_ref.at[lane, dst_slot],
                send_sem=ssem.at[lane], recv_sem=rsem.at[lane, dst_slot],
                device_id=(lane_dst[lane],),
                device_id_type=pl.DeviceIdType.MESH)

        def mk_cp(lane, slot, g):
            return pltpu.make_async_copy(
                recv_ref.at[lane, slot],
                out_ref.at[pl.ds(lane_row0(lane, g), mh),
                           pl.ds(lane_col0(lane), nh)],
                csem.at[lane])

        for lane in range(4):
            send_ref[lane, :, :] = mm(lane, -1)

        barrier = pltpu.get_barrier_semaphore()
        for nbr in (left, right):
            pl.semaphore_signal(barrier, inc=1, device_id=(nbr,),
                                device_id_type=pl.DeviceIdType.MESH)
        pl.semaphore_wait(barrier, 2)
        for lane in range(4):
            mk_rdma(lane, send_ref.at[lane], 0).start()

        def step(g, slot):
            prev = 1 - slot
            for lane in (0, 2, 1, 3):
                mk_rdma(lane, send_ref.at[lane], slot).wait_recv()

                @pl.when(g < N_DEV - 1)
                def _():
                    recv_ref[lane, slot, :, :] = (
                        recv_ref[lane, slot, :, :] + mm(lane, g))

                @pl.when(g == N_DEV - 2)
                def _():
                    recv_ref[lane, slot, :, :] = _gelu(
                        recv_ref[lane, slot, :, :])

                @pl.when(g >= N_DEV - 1)
                def _():
                    mk_cp(lane, prev, g - 1).wait()

                @pl.when(g >= N_DEV - 2)
                def _():
                    mk_cp(lane, slot, g).start()

                @pl.when(g < N_STEP - 1)
                def _():
                    mk_rdma(lane, recv_ref.at[lane, prev], prev).wait_send()

                @pl.when(jnp.logical_and(g >= 1, g < N_STEP - 1))
                def _():
                    pl.semaphore_signal(
                        credit.at[lane], inc=1,
                        device_id=(lane_up[lane],),
                        device_id_type=pl.DeviceIdType.MESH)
                    pl.semaphore_wait(credit.at[lane], 1)

                @pl.when(g < N_STEP - 1)
                def _():
                    mk_rdma(lane, recv_ref.at[lane, slot], prev).start()

        def pair_body(k, carry):
            step(2 * k, 0)
            step(2 * k + 1, 1)
            return carry

        lax.fori_loop(0, N_DEV - 1, pair_body, 0)

        for lane in range(4):
            mk_rdma(lane, recv_ref.at[lane, 1], 1).wait_send()
            mk_cp(lane, 1, N_STEP - 1).wait()

    return pl.pallas_call(
        body,
        out_shape=jax.ShapeDtypeStruct((m, n), jnp.float32),
        in_specs=[
            pl.BlockSpec(memory_space=pltpu.VMEM),
            pl.BlockSpec(memory_space=pltpu.VMEM),
            pl.BlockSpec(memory_space=pltpu.SMEM),
        ],
        out_specs=pl.BlockSpec(memory_space=pl.ANY),
        scratch_shapes=[
            pltpu.VMEM((4, mh, nh), jnp.float32),
            pltpu.VMEM((4, 2, mh, nh), jnp.float32),
            pltpu.SemaphoreType.DMA((4,)),
            pltpu.SemaphoreType.DMA((4, 2)),
            pltpu.SemaphoreType.DMA((4,)),
            pltpu.SemaphoreType.REGULAR((4,)),
        ],
        compiler_params=pltpu.CompilerParams(collective_id=0),
    )(x, w_mat, meta)
